# Initial kernel scaffold; baseline (speedup 1.0000x reference)
import jax
import jax.numpy as jnp
from jax.experimental import pallas as pl


def kernel(gains, bws, pms, gain_table, bw_table, pm_table, W_proj, b_proj, W_fc, b_fc):
    raise NotImplementedError("write your pallas kernel here")


# speedup vs baseline: 3.6667x; 3.6667x over previous
import jax
import jax.numpy as jnp
from jax import lax
from jax.experimental import pallas as pl
from jax.experimental.pallas import tpu as pltpu
from jax.experimental.pallas import tpu_sc as plsc

B = 16384
EMB = 128
LAT = 64

NC, NS = 2, 16
NW = NC * NS
ROWS_PER_W = B // NW
CHUNK = 128
NCH = ROWS_PER_W // CHUNK
NCHUNKS = 3 * NCH
DEPTH = 4

BM = 2048


def _gather_body(gidx, bidx, pidx, gt, bt, pt, o1, o2, o3,
                 idx_v, rows_v, s0, s1, s2, s3):
    sems = (s0, s1, s2, s3)
    wid = lax.axis_index("s") * NC + lax.axis_index("c")
    base = wid * ROWS_PER_W
    tbls = (gt, bt, pt)
    outs = (o1, o2, o3)

    for t, idx_hbm in enumerate((gidx, bidx, pidx)):
        pltpu.sync_copy(idx_hbm.at[wid], idx_v.at[pl.ds(t * NCH, NCH)])

    descs = [None] * NCHUNKS

    def fire(c):
        slot = c % DEPTH
        descs[c] = pltpu.async_copy(
            tbls[c // NCH].at[idx_v.at[c]], rows_v.at[slot], sems[slot])

    for c in range(DEPTH):
        fire(c)
    for c in range(NCHUNKS):
        descs[c].wait()
        t, j = divmod(c, NCH)
        pltpu.sync_copy(rows_v.at[c % DEPTH],
                        outs[t].at[pl.ds(base + j * CHUNK, CHUNK)])
        if c + DEPTH < NCHUNKS:
            fire(c + DEPTH)


def _mlp_body(g, bw, p, wp, bp, wf, bfc, o):
    wfv = wf[...]
    m1 = jnp.dot(wp[0 * EMB:1 * EMB, :], wfv,
                 preferred_element_type=jnp.float32)
    m2 = jnp.dot(wp[1 * EMB:2 * EMB, :], wfv,
                 preferred_element_type=jnp.float32)
    m3 = jnp.dot(wp[2 * EMB:3 * EMB, :], wfv,
                 preferred_element_type=jnp.float32)
    c = jnp.dot(bp[...], wfv, preferred_element_type=jnp.float32) + bfc[...]
    acc = jnp.dot(g[...], m1, preferred_element_type=jnp.float32)
    acc += jnp.dot(bw[...], m2, preferred_element_type=jnp.float32)
    acc += jnp.dot(p[...], m3, preferred_element_type=jnp.float32)
    o[...] = acc + c


def kernel(gains, bws, pms, gain_table, bw_table, pm_table,
           W_proj, b_proj, W_fc, b_fc):
    gidx = gains.astype(jnp.int32).reshape(NW, NCH, CHUNK)
    bidx = bws.astype(jnp.int32).reshape(NW, NCH, CHUNK)
    pidx = pms.astype(jnp.int32).reshape(NW, NCH, CHUNK)

    gather = pl.kernel(
        _gather_body,
        mesh=plsc.VectorSubcoreMesh(core_axis_name="c", subcore_axis_name="s"),
        out_type=[jax.ShapeDtypeStruct((B, EMB), jnp.float32)] * 3,
        scratch_types=[
            pltpu.VMEM((NCHUNKS, CHUNK), jnp.int32),
            pltpu.VMEM((DEPTH, CHUNK, EMB), jnp.float32),
        ] + [pltpu.SemaphoreType.DMA] * DEPTH,
    )
    ge, be, pe = gather(gidx, bidx, pidx, gain_table, bw_table, pm_table)

    out = pl.pallas_call(
        _mlp_body,
        grid=(B // BM,),
        in_specs=[
            pl.BlockSpec((BM, EMB), lambda i: (i, 0)),
            pl.BlockSpec((BM, EMB), lambda i: (i, 0)),
            pl.BlockSpec((BM, EMB), lambda i: (i, 0)),
            pl.BlockSpec((3 * EMB, EMB), lambda i: (0, 0)),
            pl.BlockSpec((1, EMB), lambda i: (0, 0)),
            pl.BlockSpec((EMB, LAT), lambda i: (0, 0)),
            pl.BlockSpec((1, LAT), lambda i: (0, 0)),
        ],
        out_specs=pl.BlockSpec((BM, LAT), lambda i: (i, 0)),
        out_shape=jax.ShapeDtypeStruct((B, LAT), jnp.float32),
    )(ge, be, pe, W_proj, b_proj.reshape(1, EMB), W_fc, b_fc.reshape(1, LAT))
    return out


# speedup vs baseline: 4.1767x; 1.1391x over previous
import jax
import jax.numpy as jnp
from jax import lax
from jax.experimental import pallas as pl
from jax.experimental.pallas import tpu as pltpu
from jax.experimental.pallas import tpu_sc as plsc

B = 16384
EMB = 128
LAT = 64

NC, NS = 2, 16
NW = NC * NS
ROWS_PER_W = B // NW
CHUNK = 128
NCH = ROWS_PER_W // CHUNK
NCHUNKS = 3 * NCH
SLOTS = 7
WINDOW = 4

BM = 2048


def _gather_body(gidx, bidx, pidx, gt, bt, pt, o1, o2, o3,
                 idx_v, rows_v, *sems):
    gsems, wsems = sems[:SLOTS], sems[SLOTS:]
    wid = lax.axis_index("s") * NC + lax.axis_index("c")
    base = wid * ROWS_PER_W
    tbls = (gt, bt, pt)
    outs = (o1, o2, o3)

    for t, idx_hbm in enumerate((gidx, bidx, pidx)):
        pltpu.sync_copy(idx_hbm.at[wid], idx_v.at[pl.ds(t * NCH, NCH)])

    gdescs = [None] * NCHUNKS
    wdescs = [None] * NCHUNKS

    def fire(c):
        slot = c % SLOTS
        if c >= SLOTS:
            wdescs[c - SLOTS].wait()
        gdescs[c] = pltpu.async_copy(
            tbls[c // NCH].at[idx_v.at[c]], rows_v.at[slot], gsems[slot])

    for c in range(WINDOW):
        fire(c)
    for c in range(NCHUNKS):
        gdescs[c].wait()
        t, j = divmod(c, NCH)
        slot = c % SLOTS
        wdescs[c] = pltpu.async_copy(
            rows_v.at[slot], outs[t].at[pl.ds(base + j * CHUNK, CHUNK)],
            wsems[slot])
        if c + WINDOW < NCHUNKS:
            fire(c + WINDOW)
    for c in range(NCHUNKS - SLOTS, NCHUNKS):
        if wdescs[c] is not None:
            wdescs[c].wait()


def _mlp_body(g, bw, p, wp, bp, wf, bfc, o):
    wfv = wf[...]
    m1 = jnp.dot(wp[0 * EMB:1 * EMB, :], wfv,
                 preferred_element_type=jnp.float32)
    m2 = jnp.dot(wp[1 * EMB:2 * EMB, :], wfv,
                 preferred_element_type=jnp.float32)
    m3 = jnp.dot(wp[2 * EMB:3 * EMB, :], wfv,
                 preferred_element_type=jnp.float32)
    dn = (((0,), (1,)), ((), ()))
    acc = lax.dot_general(m1, g[...], dn, preferred_element_type=jnp.float32)
    acc += lax.dot_general(m2, bw[...], dn, preferred_element_type=jnp.float32)
    acc += lax.dot_general(m3, p[...], dn, preferred_element_type=jnp.float32)
    ct = lax.dot_general(wfv, bp[...], dn,
                         preferred_element_type=jnp.float32)
    o[...] = acc + (ct + bfc[...].reshape(LAT, 1))


def kernel(gains, bws, pms, gain_table, bw_table, pm_table,
           W_proj, b_proj, W_fc, b_fc):
    gidx = gains.astype(jnp.int32).reshape(NW, NCH, CHUNK)
    bidx = bws.astype(jnp.int32).reshape(NW, NCH, CHUNK)
    pidx = pms.astype(jnp.int32).reshape(NW, NCH, CHUNK)

    gather = pl.kernel(
        _gather_body,
        mesh=plsc.VectorSubcoreMesh(core_axis_name="c", subcore_axis_name="s"),
        out_type=[jax.ShapeDtypeStruct((B, EMB), jnp.float32)] * 3,
        scratch_types=[
            pltpu.VMEM((NCHUNKS, CHUNK), jnp.int32),
            pltpu.VMEM((SLOTS, CHUNK, EMB), jnp.float32),
        ] + [pltpu.SemaphoreType.DMA] * (2 * SLOTS),
    )
    ge, be, pe = gather(gidx, bidx, pidx, gain_table, bw_table, pm_table)

    out_t = pl.pallas_call(
        _mlp_body,
        grid=(B // BM,),
        in_specs=[
            pl.BlockSpec((BM, EMB), lambda i: (i, 0)),
            pl.BlockSpec((BM, EMB), lambda i: (i, 0)),
            pl.BlockSpec((BM, EMB), lambda i: (i, 0)),
            pl.BlockSpec((3 * EMB, EMB), lambda i: (0, 0)),
            pl.BlockSpec((1, EMB), lambda i: (0, 0)),
            pl.BlockSpec((EMB, LAT), lambda i: (0, 0)),
            pl.BlockSpec((1, LAT), lambda i: (0, 0)),
        ],
        out_specs=pl.BlockSpec((LAT, BM), lambda i: (0, i)),
        out_shape=jax.ShapeDtypeStruct((LAT, B), jnp.float32),
    )(ge, be, pe, W_proj, b_proj.reshape(1, EMB), W_fc, b_fc.reshape(1, LAT))
    return out_t.T
